# baseline (speedup 1.0000x reference)
import functools

import jax
import jax.numpy as jnp
from jax import lax
from jax.experimental import pallas as pl
from jax.experimental.pallas import tpu as pltpu
from jax.experimental.pallas import tpu_sc as plsc

_NC, _NS, _VEC = 2, 16, 16
_NW = _NC * _NS
_CH = 128
_SCH = 2 * _CH


def _comb_body(pos_ref, tab_ref, out_ref):
    out_ref[0] = pos_ref[...] + tab_ref[0, :][None, :]
    out_ref[1] = pos_ref[...] + tab_ref[1, :][None, :]


def _build_comb(pos, mask_table, t, c):
    return pl.pallas_call(
        _comb_body,
        in_specs=[
            pl.BlockSpec((t, c), lambda: (0, 0)),
            pl.BlockSpec((2, c), lambda: (0, 0)),
        ],
        out_specs=pl.BlockSpec((2, t, c), lambda: (0, 0, 0)),
        out_shape=jax.ShapeDtypeStruct((2, t, c), jnp.float32),
    )(pos, mask_table)


def _make_sc_gather(tok, t, c):
    per_w = tok // _NW
    nch = per_w // _CH
    nsc = per_w // _SCH
    mesh = plsc.VectorSubcoreMesh(
        core_axis_name="c", subcore_axis_name="s",
        num_cores=_NC, num_subcores=_NS,
    )

    @functools.partial(
        pl.kernel,
        out_type=jax.ShapeDtypeStruct((tok, c), jnp.float32),
        mesh=mesh,
        scratch_types=[
            pltpu.VMEM_SHARED((2 * t, c), jnp.float32),
            pltpu.VMEM((per_w,), jnp.int32),
            pltpu.VMEM((nch, _CH), jnp.int32),
            pltpu.VMEM((2, _SCH, c), jnp.float32),
            pltpu.SemaphoreType.DMA,
            pltpu.SemaphoreType.DMA,
            pltpu.SemaphoreType.DMA,
            pltpu.SemaphoreType.DMA,
        ],
    )
    def sc_gather(comb_hbm, mask_hbm, out_hbm,
                  comb_sh, mask_v, idx_v, bufs, g0, g1, o0, o1):
        sid = lax.axis_index("s")
        wid = sid * _NC + lax.axis_index("c")
        base = wid * per_w

        @pl.when(sid == 0)
        def _():
            pltpu.sync_copy(comb_hbm, comb_sh)

        pltpu.sync_copy(mask_hbm.at[pl.ds(base, per_w)], mask_v)

        lanes = lax.iota(jnp.int32, _VEC)

        def idx_row(j):
            def idx_vec(v, _):
                p = j * _CH + v * _VEC
                m = mask_v[pl.ds(p, _VEC)]
                tpos = lax.rem(base + p + lanes, t)
                idx_v[j, pl.ds(v * _VEC, _VEC)] = m * t + tpos
                return 0
            lax.fori_loop(0, _CH // _VEC, idx_vec, 0)

        def idx_super(s):
            idx_row(2 * s)
            idx_row(2 * s + 1)

        idx_super(0)
        idx_super(1)
        plsc.subcore_barrier()

        b0 = bufs.at[0]
        b1 = bufs.at[1]
        out_proto = out_hbm.at[pl.ds(0, _SCH)]

        def fire(s, buf, sem):
            pltpu.async_copy(comb_sh.at[idx_v.at[2 * s]],
                             buf.at[pl.ds(0, _CH)], sem)
            pltpu.async_copy(comb_sh.at[idx_v.at[2 * s + 1]],
                             buf.at[pl.ds(_CH, _CH)], sem)

        def gather_done(buf, sem):
            pltpu.make_async_copy(out_proto, buf, sem).wait()

        def scatter(s, buf, sem):
            pltpu.async_copy(buf, out_hbm.at[pl.ds(base + s * _SCH, _SCH)], sem)

        def scatter_done(buf, sem):
            pltpu.make_async_copy(buf, out_proto, sem).wait()

        fire(0, b0, g0)
        fire(1, b1, g1)

        def pair(g, _):
            s0 = 2 * g
            s1 = s0 + 1

            @pl.when(s0 + 2 < nsc)
            def _():
                idx_super(s0 + 2)

            @pl.when(s1 + 2 < nsc)
            def _():
                idx_super(s1 + 2)

            gather_done(b0, g0)
            scatter(s0, b0, o0)
            gather_done(b1, g1)
            scatter(s1, b1, o1)

            scatter_done(b0, o0)

            @pl.when(s0 + 2 < nsc)
            def _():
                fire(s0 + 2, b0, g0)

            scatter_done(b1, o1)

            @pl.when(s1 + 2 < nsc)
            def _():
                fire(s1 + 2, b1, g1)
            return 0

        lax.fori_loop(0, nsc // 2, pair, 0)

        gather_done(b0, g0)
        scatter(nsc - 1, b0, o0)
        scatter_done(b0, o0)

    return sc_gather


def kernel(x, mask, pos_emb, mask_table):
    b, t, c = x.shape
    tok = b * t
    pos = pos_emb[0, :t, :]
    m_flat = mask.astype(jnp.int32).reshape(tok)
    comb = _build_comb(pos, mask_table, t, c).reshape(2 * t, c)
    out = _make_sc_gather(tok, t, c)(comb, m_flat)
    return out.reshape(b, t, c)


# speedup vs baseline: 1.2956x; 1.2956x over previous
import functools

import jax
import jax.numpy as jnp
from jax import lax
from jax.experimental import pallas as pl
from jax.experimental.pallas import tpu as pltpu
from jax.experimental.pallas import tpu_sc as plsc

_NC, _NS, _VEC = 2, 16, 16
_NW = _NC * _NS
_CH = 128
_SCH = 2 * _CH


def _comb_body(pos_ref, tab_ref, out_ref):
    out_ref[0] = pos_ref[...] + tab_ref[0, :][None, :]
    out_ref[1] = pos_ref[...] + tab_ref[1, :][None, :]


def _build_comb(pos, mask_table, t, c):
    return pl.pallas_call(
        _comb_body,
        in_specs=[
            pl.BlockSpec((t, c), lambda: (0, 0)),
            pl.BlockSpec((2, c), lambda: (0, 0)),
        ],
        out_specs=pl.BlockSpec((2, t, c), lambda: (0, 0, 0)),
        out_shape=jax.ShapeDtypeStruct((2, t, c), jnp.float32),
    )(pos, mask_table)


def _make_sc_gather(tok, t, c):
    per_w = tok // _NW
    nch = per_w // _CH
    nsc = per_w // _SCH
    mesh = plsc.VectorSubcoreMesh(
        core_axis_name="c", subcore_axis_name="s",
        num_cores=_NC, num_subcores=_NS,
    )

    @functools.partial(
        pl.kernel,
        out_type=jax.ShapeDtypeStruct((tok, c), jnp.float32),
        mesh=mesh,
        scratch_types=[
            pltpu.VMEM_SHARED((2 * t, c), jnp.float32),
            pltpu.VMEM((per_w,), jnp.int32),
            pltpu.VMEM((nch, _CH), jnp.int32),
            pltpu.VMEM((2, _SCH, c), jnp.float32),
            pltpu.SemaphoreType.DMA,
            pltpu.SemaphoreType.DMA,
        ],
    )
    def sc_gather(comb_hbm, mask_hbm, out_hbm,
                  comb_sh, mask_v, idx_v, bufs, sem0, sem1):
        sid = lax.axis_index("s")
        wid = sid * _NC + lax.axis_index("c")
        base = wid * per_w

        @pl.when(sid == 0)
        def _():
            pltpu.sync_copy(comb_hbm, comb_sh)

        pltpu.sync_copy(mask_hbm.at[pl.ds(base, per_w)], mask_v)

        lanes = lax.iota(jnp.int32, _VEC)

        def idx_row(j, _):
            def idx_vec(v, _):
                p = j * _CH + v * _VEC
                m = mask_v[pl.ds(p, _VEC)]
                tpos = lax.rem(base + p + lanes, t)
                idx_v[j, pl.ds(v * _VEC, _VEC)] = m * t + tpos
                return 0
            return lax.fori_loop(0, _CH // _VEC, idx_vec, 0)

        lax.fori_loop(0, 4, idx_row, 0)
        plsc.subcore_barrier()

        b0 = bufs.at[0]
        b1 = bufs.at[1]

        def fire(s, buf, sem):
            pltpu.async_copy(comb_sh.at[idx_v.at[2 * s]],
                             buf.at[pl.ds(0, _CH)], sem)
            pltpu.async_copy(comb_sh.at[idx_v.at[2 * s + 1]],
                             buf.at[pl.ds(_CH, _CH)], sem)

        def drain(buf, sem):
            pltpu.make_async_copy(out_hbm.at[pl.ds(0, _SCH)], buf, sem).wait()

        def scatter(s, buf):
            pltpu.sync_copy(buf, out_hbm.at[pl.ds(base + s * _SCH, _SCH)])

        fire(0, b0, sem0)
        fire(1, b1, sem1)

        lax.fori_loop(4, nch, idx_row, 0)

        def pair(g, _):
            s0 = 2 * g
            s1 = s0 + 1
            drain(b0, sem0)
            scatter(s0, b0)

            @pl.when(s0 + 2 < nsc)
            def _():
                fire(s0 + 2, b0, sem0)

            drain(b1, sem1)
            scatter(s1, b1)

            @pl.when(s1 + 2 < nsc)
            def _():
                fire(s1 + 2, b1, sem1)
            return 0

        lax.fori_loop(0, nsc // 2, pair, 0)

        drain(b0, sem0)
        scatter(nsc - 1, b0)

    return sc_gather


def kernel(x, mask, pos_emb, mask_table):
    b, t, c = x.shape
    tok = b * t
    pos = pos_emb[0, :t, :]
    m_flat = mask.astype(jnp.int32).reshape(tok)
    comb = _build_comb(pos, mask_table, t, c).reshape(2 * t, c)
    out = _make_sc_gather(tok, t, c)(comb, m_flat)
    return out.reshape(b, t, c)
